# baseline (speedup 1.0000x reference)
import functools

import jax
import jax.numpy as jnp
from jax import lax
from jax.experimental import pallas as pl
from jax.experimental.pallas import tpu as pltpu
from jax.experimental.pallas import tpu_sc as plsc

NC = 2
NS = 16
NW = NC * NS

C = 128
R = 4


def _gather_kernel(n_total, d):
    b_per_w = n_total // NW
    t_steps = b_per_w // C
    assert t_steps % R == 0

    mesh = plsc.VectorSubcoreMesh(
        core_axis_name="c", subcore_axis_name="s",
        num_cores=NC, num_subcores=NS)

    @functools.partial(
        pl.kernel,
        out_type=jax.ShapeDtypeStruct((n_total, d), jnp.float32),
        mesh=mesh,
        compiler_params=pltpu.CompilerParams(use_tc_tiling_on_sc=False),
        scratch_types=[
            pltpu.VMEM((t_steps, C), jnp.int32),
            pltpu.VMEM((R, C, d), jnp.float32),
            [pltpu.SemaphoreType.DMA] * R,
            [pltpu.SemaphoreType.DMA] * R,
        ],
    )
    def body(table_hbm, idx_hbm, out_hbm, idx_v, rows_v, gsems, ssems):
        wid = lax.axis_index("s") * NC + lax.axis_index("c")
        base = wid * b_per_w

        pltpu.sync_copy(idx_hbm.at[wid], idx_v)

        def gather_start(b, t):
            pltpu.async_copy(table_hbm.at[idx_v.at[t]], rows_v.at[b], gsems[b])

        def gather_wait(b):
            pltpu.make_async_copy(
                table_hbm.at[idx_v.at[0]], rows_v.at[b], gsems[b]).wait()

        def store_start(b, t):
            pltpu.async_copy(
                rows_v.at[b], out_hbm.at[pl.ds(base + t * C, C)], ssems[b])

        def store_wait(b):
            pltpu.make_async_copy(
                rows_v.at[b], out_hbm.at[pl.ds(base, C)], ssems[b]).wait()

        for b in range(R):
            gather_start(b, b)

        @pl.loop(0, t_steps - R, step=R)
        def _(t0):
            for b in range(R):
                gather_wait(b)
                store_start(b, t0 + b)
            for b in range(R):
                store_wait(b)
                gather_start(b, t0 + R + b)

        for b in range(R):
            gather_wait(b)
            store_start(b, t_steps - R + b)
        for b in range(R):
            store_wait(b)

    return body


def kernel(x, table):
    batch, hist = x.shape
    vocab, d = table.shape
    n_total = batch * hist
    idx = x.reshape(NW, (n_total // NW) // C, C).astype(jnp.int32)
    out = _gather_kernel(n_total, d)(table, idx)
    return out.reshape(batch, hist, d)
